# baseline (speedup 1.0000x reference)
import functools

import jax
import jax.numpy as jnp
from jax import lax
from jax.experimental import pallas as pl
from jax.experimental.pallas import tpu as pltpu
from jax.experimental.pallas import tpu_sc as plsc

B = 16384
IN_C = 67
OUT_C = 73
SKIP = 16
IN_R = IN_C - SKIP
OUT_R = OUT_C - SKIP
NW = 32
CPW = B // NW
L = 16
NQ = 4
QW = CPW // NQ


def _sc_body(elev_t_hbm, tab_hbm, out_t_hbm, in_v, out_v, tab_v,
             sem_in, sem_out):
    wid = lax.axis_index("s") * 2 + lax.axis_index("c")
    base = wid * CPW

    def in_cp(q):
        return pltpu.make_async_copy(
            elev_t_hbm.at[pl.ds(SKIP, IN_R), pl.ds(base + q * QW, QW)],
            in_v.at[:, pl.ds(q * QW, QW)], sem_in)

    def out_cp(q):
        return pltpu.make_async_copy(
            out_v.at[:, pl.ds(q * QW, QW)],
            out_t_hbm.at[pl.ds(SKIP, OUT_R), pl.ds(base + q * QW, QW)],
            sem_out)

    cp_pos = pltpu.make_async_copy(
        elev_t_hbm.at[pl.ds(0, SKIP), pl.ds(base, CPW)],
        out_t_hbm.at[pl.ds(0, SKIP), pl.ds(base, CPW)], sem_out)
    cp_pos.start()
    for q in range(NQ):
        in_cp(q).start()
    pltpu.sync_copy(tab_hbm, tab_v)

    def quarter(q, carry):
        in_cp(q).wait()

        @plsc.parallel_loop(0, QW // L, unroll=4)
        def chunk(j):
            sl = pl.ds(q * QW + j * L, L)
            idx = in_v[2, sl].astype(jnp.int32)
            out_v[0, sl] = in_v[1, sl]
            for k in range(1, 49):
                out_v[k, sl] = in_v[k + 2, sl]
            for e in range(8):
                out_v[49 + e, sl] = plsc.load_gather(
                    tab_v, [idx, jnp.full((L,), e, jnp.int32)])

        out_cp(q).start()
        return carry

    lax.fori_loop(0, NQ, quarter, 0)
    cp_pos.wait()
    for q in range(NQ):
        out_cp(q).wait()


_sc_kernel = functools.partial(
    pl.kernel,
    out_type=jax.ShapeDtypeStruct((OUT_C, B), jnp.float32),
    mesh=plsc.VectorSubcoreMesh(core_axis_name="c", subcore_axis_name="s"),
    compiler_params=pltpu.CompilerParams(
        needs_layout_passes=False, use_tc_tiling_on_sc=True),
    scratch_types=[
        pltpu.VMEM((IN_R, CPW), jnp.float32),
        pltpu.VMEM((OUT_R, CPW), jnp.float32),
        pltpu.VMEM((4, 8), jnp.float32),
        pltpu.SemaphoreType.DMA,
        pltpu.SemaphoreType.DMA,
    ],
)(_sc_body)


@jax.jit
def kernel(elev_info, door_table, srv_dir_table):
    del srv_dir_table
    out_t = _sc_kernel(elev_info.T, door_table)
    return out_t.T


# speedup vs baseline: 1.9186x; 1.9186x over previous
import functools

import jax
import jax.numpy as jnp
from jax import lax
from jax.experimental import pallas as pl
from jax.experimental.pallas import tpu as pltpu
from jax.experimental.pallas import tpu_sc as plsc

B = 16384
IN_C = 67
OUT_C = 73
NW = 32
CPW = B // NW
L = 16


NQ = 4
QW = CPW // NQ


def _sc_body(elev_t_hbm, tab_hbm, out_t_hbm, in_v, out_v, tab_v,
             sem_in, sem_out):
    wid = lax.axis_index("s") * 2 + lax.axis_index("c")
    base = wid * CPW

    def in_cp(q):
        return pltpu.make_async_copy(
            elev_t_hbm.at[:, pl.ds(base + q * QW, QW)],
            in_v.at[:, pl.ds(q * QW, QW)], sem_in)

    def out_cp(q):
        return pltpu.make_async_copy(
            out_v.at[:, pl.ds(q * QW, QW)],
            out_t_hbm.at[:, pl.ds(base + q * QW, QW)], sem_out)

    for q in range(NQ):
        in_cp(q).start()
    pltpu.sync_copy(tab_hbm, tab_v)

    def quarter(q, carry):
        in_cp(q).wait()

        @plsc.parallel_loop(0, QW // L, unroll=4)
        def chunk(j):
            sl = pl.ds(q * QW + j * L, L)
            idx = in_v[18, sl].astype(jnp.int32)
            for c in range(16):
                out_v[c, sl] = in_v[c, sl]
            out_v[16, sl] = in_v[17, sl]
            for c in range(17, 65):
                out_v[c, sl] = in_v[c + 2, sl]
            for e in range(8):
                out_v[65 + e, sl] = plsc.load_gather(
                    tab_v, [idx, jnp.full((L,), e, jnp.int32)])

        out_cp(q).start()
        return carry

    lax.fori_loop(0, NQ, quarter, 0)
    for q in range(NQ):
        out_cp(q).wait()


_sc_kernel = functools.partial(
    pl.kernel,
    out_type=jax.ShapeDtypeStruct((OUT_C, B), jnp.float32),
    mesh=plsc.VectorSubcoreMesh(core_axis_name="c", subcore_axis_name="s"),
    compiler_params=pltpu.CompilerParams(
        needs_layout_passes=False, use_tc_tiling_on_sc=True),
    scratch_types=[
        pltpu.VMEM((IN_C, CPW), jnp.float32),
        pltpu.VMEM((OUT_C, CPW), jnp.float32),
        pltpu.VMEM((4, 8), jnp.float32),
        pltpu.SemaphoreType.DMA,
        pltpu.SemaphoreType.DMA,
    ],
)(_sc_body)


@jax.jit
def kernel(elev_info, door_table, srv_dir_table):
    del srv_dir_table
    out_t = _sc_kernel(elev_info.T, door_table)
    return out_t.T
